# baseline (speedup 1.0000x reference)
import functools

import jax
import jax.numpy as jnp
from jax import lax
from jax.experimental import pallas as pl
from jax.experimental.pallas import tpu as pltpu
from jax.experimental.pallas import tpu_sc as plsc

V = 1000000
E = 64
H = 64
B = 4096
S = 20

NC = 2
NS = 16
NW = NC * NS
N = B * S
ROWS = 128
PER_W = N // NW
CH = PER_W // ROWS
NBUF = 4

BT = 512


def _gather_body(idx_hbm, table_hbm, out_hbm, idx_v, *scratch):
    bufs = scratch[:NBUF]
    gsems = scratch[NBUF:2 * NBUF]
    ssems = scratch[2 * NBUF:3 * NBUF]
    wid = lax.axis_index("s") * NC + lax.axis_index("c")
    base = wid * PER_W
    pltpu.sync_copy(idx_hbm.at[wid], idx_v)
    gops = [None] * CH
    sops = [None] * CH

    def gather(j):
        slot = j % NBUF
        gops[j] = pltpu.async_copy(table_hbm.at[idx_v.at[j]], bufs[slot],
                                   gsems[slot])

    def store(j):
        slot = j % NBUF
        sops[j] = pltpu.async_copy(bufs[slot],
                                   out_hbm.at[pl.ds(base + j * ROWS, ROWS)],
                                   ssems[slot])

    for j in range(NBUF - 1):
        gather(j)
    for j in range(CH):
        nj = j + NBUF - 1
        if nj < CH:
            if nj >= NBUF:
                sops[nj - NBUF].wait()
            gather(nj)
        gops[j].wait()
        store(j)
    for j in range(max(0, CH - NBUF), CH):
        sops[j].wait()


def _make_gather():
    mesh = plsc.VectorSubcoreMesh(core_axis_name="c", subcore_axis_name="s")
    scratch = [pltpu.VMEM((CH, ROWS), jnp.int32)]
    scratch += [pltpu.VMEM((ROWS, E), jnp.float32) for _ in range(NBUF)]
    scratch += [pltpu.SemaphoreType.DMA for _ in range(2 * NBUF)]
    return pl.kernel(
        _gather_body,
        out_type=jax.ShapeDtypeStruct((N, E), jnp.float32),
        scratch_types=scratch,
        mesh=mesh,
        compiler_params=pltpu.CompilerParams(use_tc_tiling_on_sc=False),
    )


def _gru_body(x_ref, wih_ref, whh_ref, bih_ref, bhh_ref, out_ref, hid_ref):
    wih = wih_ref[...]
    whh = whh_ref[...]
    bih = bih_ref[...]
    bhh = bhh_ref[...]
    h = jnp.zeros((BT, H), jnp.float32)
    for t in range(S):
        xt = x_ref[:, t, :]
        gi = jnp.dot(xt, wih, preferred_element_type=jnp.float32) + bih
        gh = jnp.dot(h, whh, preferred_element_type=jnp.float32) + bhh
        r = jax.nn.sigmoid(gi[:, :H] + gh[:, :H])
        z = jax.nn.sigmoid(gi[:, H:2 * H] + gh[:, H:2 * H])
        n = jnp.tanh(gi[:, 2 * H:] + r * gh[:, 2 * H:])
        h = (1.0 - z) * n + z * h
        out_ref[:, t, :] = h
    hid_ref[...] = h


def _gru_call(x3, wih_t, whh_t, bih2, bhh2):
    return pl.pallas_call(
        _gru_body,
        grid=(B // BT,),
        in_specs=[
            pl.BlockSpec((BT, S, E), lambda i: (i, 0, 0)),
            pl.BlockSpec((E, 3 * H), lambda i: (0, 0)),
            pl.BlockSpec((H, 3 * H), lambda i: (0, 0)),
            pl.BlockSpec((1, 3 * H), lambda i: (0, 0)),
            pl.BlockSpec((1, 3 * H), lambda i: (0, 0)),
        ],
        out_specs=[
            pl.BlockSpec((BT, S, H), lambda i: (i, 0, 0)),
            pl.BlockSpec((BT, H), lambda i: (i, 0)),
        ],
        out_shape=[
            jax.ShapeDtypeStruct((B, S, H), jnp.float32),
            jax.ShapeDtypeStruct((B, H), jnp.float32),
        ],
        compiler_params=pltpu.CompilerParams(
            dimension_semantics=("parallel",)),
    )(x3, wih_t, whh_t, bih2, bhh2)


def kernel(input, emb, W_ih, W_hh, b_ih, b_hh):
    idx3 = input.astype(jnp.int32).reshape(NW, CH, ROWS)
    x = _make_gather()(idx3, emb)
    x3 = x.reshape(B, S, E)
    out, hid = _gru_call(x3, W_ih.T, W_hh.T, b_ih[None], b_hh[None])
    return (out, hid[None])


# speedup vs baseline: 1.0829x; 1.0829x over previous
import functools

import jax
import jax.numpy as jnp
from jax import lax
from jax.experimental import pallas as pl
from jax.experimental.pallas import tpu as pltpu
from jax.experimental.pallas import tpu_sc as plsc

V = 1000000
E = 64
H = 64
B = 4096
S = 20

NC = 2
NS = 16
NW = NC * NS
N = B * S
ROWS = 128
PER_W = N // NW
CH = PER_W // ROWS
NBUF = 4

BT = 512


def _gather_body(idx_hbm, table_hbm, out_hbm, idx_v, *scratch):
    bufs = scratch[:NBUF]
    gsems = scratch[NBUF:2 * NBUF]
    ssems = scratch[2 * NBUF:3 * NBUF]
    wid = lax.axis_index("s") * NC + lax.axis_index("c")
    base = wid * PER_W
    pltpu.sync_copy(idx_hbm.at[wid], idx_v)
    gops = [None] * CH
    sops = [None] * CH

    def gather(j):
        slot = j % NBUF
        gops[j] = pltpu.async_copy(table_hbm.at[idx_v.at[j]], bufs[slot],
                                   gsems[slot])

    def store(j):
        slot = j % NBUF
        sops[j] = pltpu.async_copy(bufs[slot],
                                   out_hbm.at[pl.ds(base + j * ROWS, ROWS)],
                                   ssems[slot])

    for j in range(NBUF - 1):
        gather(j)
    for j in range(CH):
        nj = j + NBUF - 1
        if nj < CH:
            if nj >= NBUF:
                sops[nj - NBUF].wait()
            gather(nj)
        gops[j].wait()
        store(j)
    for j in range(max(0, CH - NBUF), CH):
        sops[j].wait()


def _make_gather():
    mesh = plsc.VectorSubcoreMesh(core_axis_name="c", subcore_axis_name="s")
    scratch = [pltpu.VMEM((CH, ROWS), jnp.int32)]
    scratch += [pltpu.VMEM((ROWS, E), jnp.float32) for _ in range(NBUF)]
    scratch += [pltpu.SemaphoreType.DMA for _ in range(2 * NBUF)]
    return pl.kernel(
        _gather_body,
        out_type=jax.ShapeDtypeStruct((N, E), jnp.float32),
        scratch_types=scratch,
        mesh=mesh,
        compiler_params=pltpu.CompilerParams(use_tc_tiling_on_sc=False),
    )


def _gru_body(x_ref, wxr_ref, wxz_ref, wxn_ref, whr_ref, whz_ref, whn_ref,
              brz_ref, bn1_ref, bn2_ref, out_ref, hid_ref):
    wxr = wxr_ref[...]
    wxz = wxz_ref[...]
    wxn = wxn_ref[...]
    whr = whr_ref[...]
    whz = whz_ref[...]
    whn = whn_ref[...]
    br = brz_ref[0:1, :]
    bz = brz_ref[1:2, :]
    bn1 = bn1_ref[...]
    bn2 = bn2_ref[...]
    h = jnp.zeros((BT, H), jnp.float32)
    for t in range(S):
        xt = x_ref[:, t, :]
        xr = jnp.dot(xt, wxr, preferred_element_type=jnp.float32)
        xz = jnp.dot(xt, wxz, preferred_element_type=jnp.float32)
        xn = jnp.dot(xt, wxn, preferred_element_type=jnp.float32)
        hr = jnp.dot(h, whr, preferred_element_type=jnp.float32)
        hz = jnp.dot(h, whz, preferred_element_type=jnp.float32)
        hn = jnp.dot(h, whn, preferred_element_type=jnp.float32)
        r = jax.nn.sigmoid(xr + hr + br)
        z = jax.nn.sigmoid(xz + hz + bz)
        n = jnp.tanh(xn + bn1 + r * (hn + bn2))
        h = (1.0 - z) * n + z * h
        out_ref[:, t, :] = h
    hid_ref[...] = h


def _gru_call(x3, wxr, wxz, wxn, whr, whz, whn, brz, bn1, bn2):
    wspec = pl.BlockSpec((H, H), lambda i: (0, 0))
    return pl.pallas_call(
        _gru_body,
        grid=(B // BT,),
        in_specs=[
            pl.BlockSpec((BT, S, E), lambda i: (i, 0, 0)),
            wspec, wspec, wspec, wspec, wspec, wspec,
            pl.BlockSpec((2, H), lambda i: (0, 0)),
            pl.BlockSpec((1, H), lambda i: (0, 0)),
            pl.BlockSpec((1, H), lambda i: (0, 0)),
        ],
        out_specs=[
            pl.BlockSpec((BT, S, H), lambda i: (i, 0, 0)),
            pl.BlockSpec((BT, H), lambda i: (i, 0)),
        ],
        out_shape=[
            jax.ShapeDtypeStruct((B, S, H), jnp.float32),
            jax.ShapeDtypeStruct((B, H), jnp.float32),
        ],
        compiler_params=pltpu.CompilerParams(
            dimension_semantics=("parallel",)),
    )(x3, wxr, wxz, wxn, whr, whz, whn, brz, bn1, bn2)


def kernel(input, emb, W_ih, W_hh, b_ih, b_hh):
    idx3 = input.astype(jnp.int32).reshape(NW, CH, ROWS)
    x = _make_gather()(idx3, emb)
    x3 = x.reshape(B, S, E)
    wxr = W_ih[:H].T
    wxz = W_ih[H:2 * H].T
    wxn = W_ih[2 * H:].T
    whr = W_hh[:H].T
    whz = W_hh[H:2 * H].T
    whn = W_hh[2 * H:].T
    brz = jnp.stack([b_ih[:H] + b_hh[:H], b_ih[H:2 * H] + b_hh[H:2 * H]])
    bn1 = b_ih[2 * H:][None]
    bn2 = b_hh[2 * H:][None]
    out, hid = _gru_call(x3, wxr, wxz, wxn, whr, whz, whn, brz, bn1, bn2)
    return (out, hid[None])
